# Initial kernel scaffold; baseline (speedup 1.0000x reference)
import functools

import jax
import jax.numpy as jnp
from jax import lax
from jax.experimental import pallas as pl
from jax.experimental.pallas import tpu as pltpu
from jax.experimental.pallas import tpu_sc as plsc

B = 16
N = 4096
C = 64
T = N // 2
R = 1024
UNM = T - R
TT = 256
NT = T // TT
DUMMY = T
ACC_ROWS = T + 8


def _tc_body(pts_ref, srcs_ref, dsts_ref, gidx_ref, sidx_ref):
    b = pl.program_id(0)

    src = pts_ref[0, :, 0:64]
    dst = pts_ref[0, :, 64:128]

    nb = jax.lax.rsqrt(jnp.sum(dst * dst, axis=1, keepdims=True))
    b_n = dst * nb

    vmax_parts = []
    nidx_parts = []
    for tt in range(NT):
        a_t = src[tt * TT:(tt + 1) * TT, :]
        na_t = jax.lax.rsqrt(jnp.sum(a_t * a_t, axis=1))
        raw = jax.lax.dot_general(
            a_t, b_n, (((1,), (1,)), ((), ())),
            preferred_element_type=jnp.float32,
            precision=jax.lax.Precision.HIGHEST)
        m = jnp.max(raw, axis=1)
        eq = raw == m[:, None]
        iot = jax.lax.broadcasted_iota(jnp.int32, (TT, T), 1)
        nidx = jnp.min(jnp.where(eq, iot, T), axis=1)
        vmax_parts.append(m * na_t)
        nidx_parts.append(nidx.astype(jnp.int32))

    vrow = jnp.concatenate([p[None, :] for p in vmax_parts], axis=1)
    jrow = jax.lax.broadcasted_iota(jnp.int32, (1, T), 1)

    rank_parts = []
    for tt in range(NT):
        vcol = vmax_parts[tt][:, None]
        icol = (jax.lax.broadcasted_iota(jnp.int32, (TT, 1), 0)
                + tt * TT)
        pred = (vrow > vcol) | ((vrow == vcol) & (jrow < icol))
        rank_parts.append(jnp.sum(pred.astype(jnp.int32), axis=1))

    sidx_parts = []
    for tt in range(NT):
        keep = rank_parts[tt] < R
        sidx_parts.append(jnp.where(keep, nidx_parts[tt], DUMMY))
    sidx_row = jnp.concatenate([p[None, :] for p in sidx_parts], axis=1)

    invc_parts = []
    for tt in range(NT):
        dcol = (jax.lax.broadcasted_iota(jnp.int32, (TT, 1), 0)
                + tt * TT)
        cnt = 1.0 + jnp.sum((sidx_row == dcol).astype(jnp.float32), axis=1)
        invc_parts.append(1.0 / cnt)
    invc_row = jnp.concatenate([p[None, :] for p in invc_parts], axis=1)

    rank_row = jnp.concatenate([p[None, :] for p in rank_parts], axis=1)
    for pt in range(NT // 2, NT):
        pcol = (jax.lax.broadcasted_iota(jnp.int32, (TT, 1), 0)
                + pt * TT)
        inv = jnp.sum(jnp.where(rank_row == pcol, jrow, 0), axis=1)
        g = inv + b * T
        gr = pt - NT // 2
        gidx_ref[0, gr * 2:(gr + 1) * 2, :] = g.reshape(2, 128)

    for tt in range(NT):
        nidx_col = nidx_parts[tt][:, None]
        w = jnp.sum(jnp.where(jrow == nidx_col, invc_row, 0.0), axis=1)
        keep = rank_parts[tt] < R
        w = jnp.where(keep, w, 1.0)
        srcs_ref[0, tt * TT:(tt + 1) * TT, :] = (
            src[tt * TT:(tt + 1) * TT, :] * w[:, None])
        dsts_ref[0, tt * TT:(tt + 1) * TT, :] = (
            dst[tt * TT:(tt + 1) * TT, :] * invc_parts[tt][:, None])
        sidx_ref[0, tt * 2:(tt + 1) * 2, :] = (
            sidx_parts[tt].astype(jnp.int32).reshape(2, 128))


def _tc_stage(pts2, interpret=False):
    return pl.pallas_call(
        _tc_body,
        grid=(B,),
        in_specs=[pl.BlockSpec((1, T, 2 * C), lambda b: (b, 0, 0))],
        out_specs=[
            pl.BlockSpec((1, T, C), lambda b: (b, 0, 0)),
            pl.BlockSpec((1, T, C), lambda b: (b, 0, 0)),
            pl.BlockSpec((1, 8, 128), lambda b: (b, 0, 0)),
            pl.BlockSpec((1, 16, 128), lambda b: (b, 0, 0)),
        ],
        out_shape=[
            jax.ShapeDtypeStruct((B, T, C), jnp.float32),
            jax.ShapeDtypeStruct((B, T, C), jnp.float32),
            jax.ShapeDtypeStruct((B, 8, 128), jnp.int32),
            jax.ShapeDtypeStruct((B, 16, 128), jnp.int32),
        ],
        interpret=interpret,
    )(pts2)


def _sc_body(srcs_hbm, dsts_hbm, gidx_hbm, sidx_hbm, out_hbm,
             gidx_v, unm_v, sidx_v, rows_v, acc_sh, sem):
    c = lax.axis_index("c")
    s = lax.axis_index("s")
    b = c * 8 + s // 2
    h = s % 2
    bb = s // 2

    @pl.when(h == 0)
    def _():
        pltpu.sync_copy(dsts_hbm.at[b], acc_sh.at[bb, pl.ds(0, T)])

    pltpu.sync_copy(gidx_hbm.at[b, pl.ds(h * 4, 4)], gidx_v)
    for j in range(4):
        pltpu.async_copy(srcs_hbm.at[gidx_v.at[j]],
                         unm_v.at[pl.ds(j * 128, 128)], sem).wait()
    pltpu.sync_copy(unm_v, out_hbm.at[pl.ds(b * 3072 + h * 512, 512)])

    pltpu.sync_copy(sidx_hbm.at[b, pl.ds(h * 8, 8)], sidx_v)
    pltpu.sync_copy(srcs_hbm.at[pl.ds(b * T + h * R, R)], rows_v)
    plsc.subcore_barrier()
    for j in range(8):
        pltpu.sync_copy(rows_v.at[pl.ds(j * 128, 128)],
                        acc_sh.at[bb].at[sidx_v.at[j]], add=True)
    plsc.subcore_barrier()

    pltpu.sync_copy(acc_sh.at[bb, pl.ds(h * R, R)], rows_v)
    pltpu.sync_copy(rows_v, out_hbm.at[pl.ds(b * 3072 + UNM + h * R, R)])


def _sc_stage(srcs_flat, dsts, gidx, sidx):
    mesh = plsc.VectorSubcoreMesh(core_axis_name="c", subcore_axis_name="s")
    return pl.kernel(
        _sc_body,
        out_type=jax.ShapeDtypeStruct((B * 3072, C), jnp.float32),
        mesh=mesh,
        scratch_types=[
            pltpu.VMEM((4, 128), jnp.int32),
            pltpu.VMEM((512, C), jnp.float32),
            pltpu.VMEM((8, 128), jnp.int32),
            pltpu.VMEM((R, C), jnp.float32),
            pltpu.VMEM_SHARED((8, ACC_ROWS, C), jnp.float32),
            pltpu.SemaphoreType.DMA,
        ],
    )(srcs_flat, dsts, gidx, sidx)


@jax.jit
def kernel(points):
    pts2 = points.reshape(B, T, 2 * C)
    srcs, dsts, gidx, sidx = _tc_stage(pts2)
    out_flat = _sc_stage(srcs.reshape(B * T, C), dsts, gidx, sidx)
    return out_flat.reshape(B, 3072, C)


# speedup vs baseline: 2.7383x; 2.7383x over previous
import jax
import jax.numpy as jnp
from jax import lax
from jax.experimental import pallas as pl
from jax.experimental.pallas import tpu as pltpu
from jax.experimental.pallas import tpu_sc as plsc

B = 16
N = 4096
C = 64
T = N // 2
R = 1024
UNM = T - R
TT = 256
NT = T // TT
DUMMY = T
ACC_ROWS = T + 8


def _tc_body(pts_ref, met_ref, unm_ref, srcs_ref, dsts_ref, sidx_ref):
    src = pts_ref[0, :, 0:64]
    dst = pts_ref[0, :, 64:128]
    b_n = met_ref[0, :, 64:128]

    vmax_parts = []
    nidx_parts = []
    for tt in range(NT):
        a_n = met_ref[0, tt * TT:(tt + 1) * TT, 0:64]
        raw = jax.lax.dot_general(
            a_n, b_n, (((1,), (1,)), ((), ())),
            preferred_element_type=jnp.float32)
        m = jnp.max(raw, axis=1)
        eq = raw == m[:, None]
        iot = jax.lax.broadcasted_iota(jnp.int32, (TT, T), 1)
        nidx = jnp.min(jnp.where(eq, iot, T), axis=1)
        vmax_parts.append(m)
        nidx_parts.append(nidx.astype(jnp.int32))

    vrow = jnp.concatenate([p[None, :] for p in vmax_parts], axis=1)
    jrow = jax.lax.broadcasted_iota(jnp.int32, (1, T), 1)

    rank_parts = []
    for tt in range(NT):
        vcol = vmax_parts[tt][:, None]
        icol = (jax.lax.broadcasted_iota(jnp.int32, (TT, 1), 0)
                + tt * TT)
        pred = (vrow > vcol) | ((vrow == vcol) & (jrow < icol))
        rank_parts.append(jnp.sum(pred.astype(jnp.int32), axis=1))
    rank_row = jnp.concatenate([p[None, :] for p in rank_parts], axis=1)

    sidx_parts = []
    for tt in range(NT):
        keep = rank_parts[tt] < R
        sidx_parts.append(jnp.where(keep, nidx_parts[tt], DUMMY))
    sidx_row = jnp.concatenate([p[None, :] for p in sidx_parts], axis=1)

    invc_parts = []
    for tt in range(NT):
        dcol = (jax.lax.broadcasted_iota(jnp.int32, (TT, 1), 0)
                + tt * TT)
        cnt = 1.0 + jnp.sum((sidx_row == dcol).astype(jnp.float32), axis=1)
        invc_parts.append(1.0 / cnt)
    invc_row = jnp.concatenate([p[None, :] for p in invc_parts], axis=1)

    for pt in range(NT // 2, NT):
        pcol = (jax.lax.broadcasted_iota(jnp.int32, (TT, 1), 0)
                + pt * TT)
        onehot = (rank_row == pcol).astype(jnp.float32)
        unm_t = jax.lax.dot_general(
            onehot, src, (((1,), (0,)), ((), ())),
            preferred_element_type=jnp.float32,
            precision=jax.lax.Precision.HIGHEST)
        gr = pt - NT // 2
        unm_ref[0, gr * TT:(gr + 1) * TT, :] = unm_t

    zpad = jnp.zeros((TT, C), jnp.float32)
    for tt in range(NT):
        nidx_col = nidx_parts[tt][:, None]
        w = jnp.sum(jnp.where(jrow == nidx_col, invc_row, 0.0), axis=1)
        keep = rank_parts[tt] < R
        w = jnp.where(keep, w, 1.0)
        srcs_ref[0, tt * TT:(tt + 1) * TT, :] = jnp.concatenate(
            [src[tt * TT:(tt + 1) * TT, :] * w[:, None], zpad], axis=1)
        dsts_ref[0, tt * TT:(tt + 1) * TT, :] = jnp.concatenate(
            [dst[tt * TT:(tt + 1) * TT, :] * invc_parts[tt][:, None], zpad],
            axis=1)
        sidx_ref[0, tt * 2:(tt + 1) * 2, :] = (
            sidx_parts[tt].astype(jnp.int32).reshape(2, 128))


def _tc_stage(pts2, met2, interpret=False):
    return pl.pallas_call(
        _tc_body,
        grid=(B,),
        in_specs=[pl.BlockSpec((1, T, 2 * C), lambda b: (b, 0, 0)),
                  pl.BlockSpec((1, T, 2 * C), lambda b: (b, 0, 0))],
        out_specs=[
            pl.BlockSpec((1, UNM, C), lambda b: (b, 0, 0)),
            pl.BlockSpec((1, T, 2 * C), lambda b: (b, 0, 0)),
            pl.BlockSpec((1, T, 2 * C), lambda b: (b, 0, 0)),
            pl.BlockSpec((1, 16, 128), lambda b: (b, 0, 0)),
        ],
        out_shape=[
            jax.ShapeDtypeStruct((B, UNM, C), jnp.float32),
            jax.ShapeDtypeStruct((B, T, 2 * C), jnp.float32),
            jax.ShapeDtypeStruct((B, T, 2 * C), jnp.float32),
            jax.ShapeDtypeStruct((B, 16, 128), jnp.int32),
        ],
        interpret=interpret,
    )(pts2, met2)


Q = 4
W = T // Q


def _sc_body(srcs_hbm, dsts_hbm, sidx_hbm, out_hbm, sidx_v, rows_v, acc_sh):
    c = lax.axis_index("c")
    s = lax.axis_index("s")
    bb = s // Q
    q = s % Q

    for rd in range(2):
        b = c * 8 + rd * 4 + bb

        pltpu.sync_copy(dsts_hbm.at[b, pl.ds(q * W, W)],
                        acc_sh.at[bb, pl.ds(q * W, W)])
        pltpu.sync_copy(sidx_hbm.at[b, pl.ds(q * 4, 4)], sidx_v)
        plsc.subcore_barrier()

        for ch in range(2):
            pltpu.sync_copy(
                srcs_hbm.at[b, pl.ds(q * W + ch * (W // 2), W // 2)], rows_v)
            for j in range(2):
                pltpu.sync_copy(rows_v.at[pl.ds(j * 128, 128)],
                                acc_sh.at[bb].at[sidx_v.at[ch * 2 + j]],
                                add=True)
        plsc.subcore_barrier()

        for ch in range(2):
            pltpu.sync_copy(
                acc_sh.at[bb, pl.ds(q * W + ch * (W // 2), W // 2)], rows_v)
            pltpu.sync_copy(
                rows_v, out_hbm.at[b, pl.ds(q * W + ch * (W // 2), W // 2)])
        if rd == 0:
            plsc.subcore_barrier()


def _sc_stage(srcs, dsts, sidx):
    mesh = plsc.VectorSubcoreMesh(core_axis_name="c", subcore_axis_name="s")
    return pl.kernel(
        _sc_body,
        out_type=jax.ShapeDtypeStruct((B, T, 2 * C), jnp.float32),
        mesh=mesh,
        scratch_types=[
            pltpu.VMEM((4, 128), jnp.int32),
            pltpu.VMEM((W // 2, 2 * C), jnp.float32),
            pltpu.VMEM_SHARED((4, ACC_ROWS, 2 * C), jnp.float32),
        ],
    )(srcs, dsts, sidx)


@jax.jit
def kernel(points):
    metric = points / jnp.linalg.norm(points, axis=-1, keepdims=True)
    pts2 = points.reshape(B, T, 2 * C)
    met2 = metric.reshape(B, T, 2 * C)
    unm, srcs, dsts, sidx = _tc_stage(pts2, met2)
    dstm = _sc_stage(srcs, dsts, sidx)
    return jnp.concatenate([unm, dstm[:, :, :C]], axis=1)


# speedup vs baseline: 3.6315x; 1.3262x over previous
import jax
import jax.numpy as jnp
from jax import lax
from jax.experimental import pallas as pl
from jax.experimental.pallas import tpu as pltpu
from jax.experimental.pallas import tpu_sc as plsc

B = 16
N = 4096
C = 64
T = N // 2
R = 1024
UNM = T - R
TT = 256
NT = T // TT
DUMMY = T
ACC_ROWS = T + 8
UDUMMY = UNM
UNM_ROWS = UNM + 8


def _tc_body(pts_ref, met_ref, srcs_ref, dsts_ref, sidx_ref, uidx_ref):
    src = pts_ref[0, :, 0:64]
    dst = pts_ref[0, :, 64:128]
    b_n = met_ref[0, :, 64:128]

    vmax_parts = []
    nidx_parts = []
    for tt in range(NT):
        a_n = met_ref[0, tt * TT:(tt + 1) * TT, 0:64]
        raw = jax.lax.dot_general(
            a_n, b_n, (((1,), (1,)), ((), ())),
            preferred_element_type=jnp.float32)
        m = jnp.max(raw, axis=1)
        nidx = jnp.argmax(raw, axis=1)
        vmax_parts.append(m)
        nidx_parts.append(nidx.astype(jnp.int32))

    vrow = jnp.concatenate([p[None, :] for p in vmax_parts], axis=1)
    jrow = jax.lax.broadcasted_iota(jnp.int32, (1, T), 1)

    rank_parts = []
    for tt in range(NT):
        vcol = vmax_parts[tt][:, None]
        icol = (jax.lax.broadcasted_iota(jnp.int32, (TT, 1), 0)
                + tt * TT)
        pred = (vrow > vcol) | ((vrow == vcol) & (jrow < icol))
        rank_parts.append(jnp.sum(pred.astype(jnp.int32), axis=1))
    rank_row = jnp.concatenate([p[None, :] for p in rank_parts], axis=1)

    sidx_parts = []
    for tt in range(NT):
        keep = rank_parts[tt] < R
        sidx_parts.append(jnp.where(keep, nidx_parts[tt], DUMMY))
    sidx_row = jnp.concatenate([p[None, :] for p in sidx_parts], axis=1)

    invc_parts = []
    for tt in range(NT):
        dcol = (jax.lax.broadcasted_iota(jnp.int32, (TT, 1), 0)
                + tt * TT)
        cnt = 1.0 + jnp.sum((sidx_row == dcol).astype(jnp.float32), axis=1)
        invc_parts.append(1.0 / cnt)
    invc_row = jnp.concatenate([p[None, :] for p in invc_parts], axis=1)

    zpad = jnp.zeros((TT, C), jnp.float32)
    for tt in range(NT):
        nidx_col = nidx_parts[tt][:, None]
        w = jnp.sum(jnp.where(jrow == nidx_col, invc_row, 0.0), axis=1)
        keep = rank_parts[tt] < R
        w = jnp.where(keep, w, 1.0)
        srcs_ref[0, tt * TT:(tt + 1) * TT, :] = jnp.concatenate(
            [src[tt * TT:(tt + 1) * TT, :] * w[:, None], zpad], axis=1)
        dsts_ref[0, tt * TT:(tt + 1) * TT, :] = jnp.concatenate(
            [dst[tt * TT:(tt + 1) * TT, :] * invc_parts[tt][:, None], zpad],
            axis=1)
        sidx_ref[0, tt * 2:(tt + 1) * 2, :] = (
            sidx_parts[tt].astype(jnp.int32).reshape(2, 128))
        uidx = jnp.where(keep, UDUMMY, rank_parts[tt] - R)
        uidx_ref[0, tt * 2:(tt + 1) * 2, :] = uidx.astype(jnp.int32).reshape(2, 128)


def _tc_stage(pts2, met2, interpret=False):
    return pl.pallas_call(
        _tc_body,
        grid=(B,),
        in_specs=[pl.BlockSpec((1, T, 2 * C), lambda b: (b, 0, 0)),
                  pl.BlockSpec((1, T, 2 * C), lambda b: (b, 0, 0))],
        out_specs=[
            pl.BlockSpec((1, T, 2 * C), lambda b: (b, 0, 0)),
            pl.BlockSpec((1, T, 2 * C), lambda b: (b, 0, 0)),
            pl.BlockSpec((1, 16, 128), lambda b: (b, 0, 0)),
            pl.BlockSpec((1, 16, 128), lambda b: (b, 0, 0)),
        ],
        out_shape=[
            jax.ShapeDtypeStruct((B, T, 2 * C), jnp.float32),
            jax.ShapeDtypeStruct((B, T, 2 * C), jnp.float32),
            jax.ShapeDtypeStruct((B, 16, 128), jnp.int32),
            jax.ShapeDtypeStruct((B, 16, 128), jnp.int32),
        ],
        interpret=interpret,
    )(pts2, met2)


Q = 4
W = T // Q


def _sc_body(srcs_hbm, dsts_hbm, sidx_hbm, uidx_hbm, out_hbm,
             sidx_v, uidx_v, rows_v, acc_sh, unm_sh):
    c = lax.axis_index("c")
    s = lax.axis_index("s")
    bb = s // Q
    q = s % Q

    for rd in range(2):
        b = c * 8 + rd * 4 + bb

        pltpu.sync_copy(dsts_hbm.at[b, pl.ds(q * W, W)],
                        acc_sh.at[bb, pl.ds(q * W, W)])
        pltpu.sync_copy(sidx_hbm.at[b, pl.ds(q * 4, 4)], sidx_v)
        pltpu.sync_copy(uidx_hbm.at[b, pl.ds(q * 4, 4)], uidx_v)
        plsc.subcore_barrier()

        for ch in range(4):
            pltpu.sync_copy(srcs_hbm.at[b, pl.ds(q * W + ch * 128, 128)],
                            rows_v)
            pltpu.sync_copy(rows_v, acc_sh.at[bb].at[sidx_v.at[ch]], add=True)
            pltpu.sync_copy(rows_v, unm_sh.at[bb].at[uidx_v.at[ch]])
        plsc.subcore_barrier()

        pltpu.sync_copy(unm_sh.at[bb, pl.ds(q * (UNM // Q), UNM // Q)],
                        out_hbm.at[b, pl.ds(q * (UNM // Q), UNM // Q)])
        pltpu.sync_copy(acc_sh.at[bb, pl.ds(q * W, W)],
                        out_hbm.at[b, pl.ds(UNM + q * W, W)])
        if rd == 0:
            plsc.subcore_barrier()


def _sc_stage(srcs, dsts, sidx, uidx):
    mesh = plsc.VectorSubcoreMesh(core_axis_name="c", subcore_axis_name="s")
    return pl.kernel(
        _sc_body,
        out_type=jax.ShapeDtypeStruct((B, UNM + T, 2 * C), jnp.float32),
        mesh=mesh,
        scratch_types=[
            pltpu.VMEM((4, 128), jnp.int32),
            pltpu.VMEM((4, 128), jnp.int32),
            pltpu.VMEM((128, 2 * C), jnp.float32),
            pltpu.VMEM_SHARED((4, ACC_ROWS, 2 * C), jnp.float32),
            pltpu.VMEM_SHARED((4, UNM_ROWS, 2 * C), jnp.float32),
        ],
    )(srcs, dsts, sidx, uidx)


@jax.jit
def kernel(points):
    metric = points / jnp.linalg.norm(points, axis=-1, keepdims=True)
    pts2 = points.reshape(B, T, 2 * C)
    met2 = metric.reshape(B, T, 2 * C)
    srcs, dsts, sidx, uidx = _tc_stage(pts2, met2)
    out2 = _sc_stage(srcs, dsts, sidx, uidx)
    return out2[:, :, :C]


# speedup vs baseline: 3.6362x; 1.0013x over previous
import jax
import jax.numpy as jnp
from jax import lax
from jax.experimental import pallas as pl
from jax.experimental.pallas import tpu as pltpu
from jax.experimental.pallas import tpu_sc as plsc

B = 16
N = 4096
C = 64
T = N // 2
R = 1024
UNM = T - R
TT = 256
NT = T // TT
OUT_ROWS = UNM + T
COMB_ROWS = OUT_ROWS + 8


def _tc_body(pts_ref, met_ref, srcs_ref, dsts_ref, cidx_ref):
    src = pts_ref[0, :, 0:64]
    dst = pts_ref[0, :, 64:128]
    b_n = met_ref[0, :, 64:128]

    ones_col = jnp.ones((T, 1), jnp.float32)

    vmax_parts = []
    nidx_parts = []
    for tt in range(NT):
        a_n = met_ref[0, tt * TT:(tt + 1) * TT, 0:64]
        raw = jax.lax.dot_general(
            a_n, b_n, (((1,), (1,)), ((), ())),
            preferred_element_type=jnp.float32)
        m = jnp.max(raw, axis=1)
        nidx = jnp.argmax(raw, axis=1)
        vmax_parts.append(m)
        nidx_parts.append(nidx.astype(jnp.int32))

    vrow = jnp.concatenate([p[None, :] for p in vmax_parts], axis=1)
    jrow = jax.lax.broadcasted_iota(jnp.int32, (1, T), 1)

    rank_parts = []
    for tt in range(NT):
        vcol = vmax_parts[tt][:, None]
        icol = (jax.lax.broadcasted_iota(jnp.int32, (TT, 1), 0)
                + tt * TT)
        pred = (vrow > vcol) | ((vrow == vcol) & (jrow < icol))
        rank_f = jax.lax.dot_general(
            pred.astype(jnp.float32), ones_col, (((1,), (0,)), ((), ())),
            preferred_element_type=jnp.float32)
        rank_parts.append(rank_f[:, 0].astype(jnp.int32))

    cidx_parts = []
    for tt in range(NT):
        keep = rank_parts[tt] < R
        cidx_parts.append(
            jnp.where(keep, UNM + nidx_parts[tt], rank_parts[tt] - R))
    cidx_row = jnp.concatenate([p[None, :] for p in cidx_parts], axis=1)

    invc_parts = []
    for tt in range(NT):
        dcol = (jax.lax.broadcasted_iota(jnp.int32, (TT, 1), 0)
                + tt * TT + UNM)
        eq_f = (cidx_row == dcol).astype(jnp.float32)
        cnt = 1.0 + jax.lax.dot_general(
            eq_f, ones_col, (((1,), (0,)), ((), ())),
            preferred_element_type=jnp.float32)[:, 0]
        invc_parts.append(1.0 / cnt)
    invc_row = jnp.concatenate([p[None, :] for p in invc_parts], axis=1)

    zpad = jnp.zeros((TT, C), jnp.float32)
    for tt in range(NT):
        nidx_col = nidx_parts[tt][:, None]
        w = jnp.sum(jnp.where(jrow == nidx_col, invc_row, 0.0), axis=1)
        keep = rank_parts[tt] < R
        w = jnp.where(keep, w, 1.0)
        srcs_ref[0, tt * TT:(tt + 1) * TT, :] = jnp.concatenate(
            [src[tt * TT:(tt + 1) * TT, :] * w[:, None], zpad], axis=1)
        dsts_ref[0, tt * TT:(tt + 1) * TT, :] = jnp.concatenate(
            [dst[tt * TT:(tt + 1) * TT, :] * invc_parts[tt][:, None], zpad],
            axis=1)
        cidx_ref[0, tt * 2:(tt + 1) * 2, :] = (
            cidx_parts[tt].astype(jnp.int32).reshape(2, 128))


def _tc_stage(pts2, met2, interpret=False):
    return pl.pallas_call(
        _tc_body,
        grid=(B,),
        in_specs=[pl.BlockSpec((1, T, 2 * C), lambda b: (b, 0, 0)),
                  pl.BlockSpec((1, T, 2 * C), lambda b: (b, 0, 0))],
        out_specs=[
            pl.BlockSpec((1, T, 2 * C), lambda b: (b, 0, 0)),
            pl.BlockSpec((1, T, 2 * C), lambda b: (b, 0, 0)),
            pl.BlockSpec((1, 16, 128), lambda b: (b, 0, 0)),
        ],
        out_shape=[
            jax.ShapeDtypeStruct((B, T, 2 * C), jnp.float32),
            jax.ShapeDtypeStruct((B, T, 2 * C), jnp.float32),
            jax.ShapeDtypeStruct((B, 16, 128), jnp.int32),
        ],
        interpret=interpret,
    )(pts2, met2)


Q = 4
W = T // Q
UQ = UNM // Q
OQ = OUT_ROWS // Q


def _sc_body(srcs_hbm, dsts_hbm, cidx_hbm, zero_hbm, out_hbm,
             cidx_v, rows_v, comb_sh):
    c = lax.axis_index("c")
    s = lax.axis_index("s")
    bb = s // Q
    q = s % Q

    for rd in range(2):
        b = c * 8 + rd * 4 + bb

        pltpu.sync_copy(zero_hbm, comb_sh.at[bb, pl.ds(q * UQ, UQ)])
        pltpu.sync_copy(dsts_hbm.at[b, pl.ds(q * W, W)],
                        comb_sh.at[bb, pl.ds(UNM + q * W, W)])
        pltpu.sync_copy(cidx_hbm.at[b, pl.ds(q * 4, 4)], cidx_v)
        plsc.subcore_barrier()

        for ch in range(4):
            pltpu.sync_copy(srcs_hbm.at[b, pl.ds(q * W + ch * 128, 128)],
                            rows_v)
            pltpu.sync_copy(rows_v, comb_sh.at[bb].at[cidx_v.at[ch]],
                            add=True)
        plsc.subcore_barrier()

        pltpu.sync_copy(comb_sh.at[bb, pl.ds(q * OQ, OQ)],
                        out_hbm.at[b, pl.ds(q * OQ, OQ)])
        if rd == 0:
            plsc.subcore_barrier()


def _sc_stage(srcs, dsts, cidx, zeros):
    mesh = plsc.VectorSubcoreMesh(core_axis_name="c", subcore_axis_name="s")
    return pl.kernel(
        _sc_body,
        out_type=jax.ShapeDtypeStruct((B, OUT_ROWS, 2 * C), jnp.float32),
        mesh=mesh,
        scratch_types=[
            pltpu.VMEM((4, 128), jnp.int32),
            pltpu.VMEM((128, 2 * C), jnp.float32),
            pltpu.VMEM_SHARED((4, COMB_ROWS, 2 * C), jnp.float32),
        ],
        compiler_params=pltpu.CompilerParams(use_tc_tiling_on_sc=True),
    )(srcs, dsts, cidx, zeros)


@jax.jit
def kernel(points):
    metric = points / jnp.linalg.norm(points, axis=-1, keepdims=True)
    pts2 = points.reshape(B, T, 2 * C)
    met2 = metric.reshape(B, T, 2 * C)
    srcs, dsts, cidx = _tc_stage(pts2, met2)
    zeros = jnp.zeros((UQ, 2 * C), jnp.float32)
    out2 = _sc_stage(srcs, dsts, cidx, zeros)
    return out2[:, :, :C]
